# baseline (speedup 1.0000x reference)
import jax
import jax.numpy as jnp
from jax.experimental import pallas as pl
from jax.experimental.pallas import tpu as pltpu

_N = 5000
_R = 50000
_C_OBJ = 151
_C_REL = 51

_BOUNDS = [3168 * i for i in range(1, 13)] + [41040, 44064, 47088]
_NO_SPLIT = 1 << 30

_CNT_BLK = 400
_SCAT_CHUNK = 5000


def _count_body(sub_idx_ref, obj_idx_ref, cnt_s_ref, cnt_o_ref):
    @pl.when(pl.program_id(0) == 0)
    def _():
        cnt_s_ref[...] = jnp.zeros_like(cnt_s_ref)
        cnt_o_ref[...] = jnp.zeros_like(cnt_o_ref)

    lanes = jax.lax.broadcasted_iota(jnp.int32, (_CNT_BLK, _N), 1)
    si = sub_idx_ref[...]
    cnt_s_ref[...] += jnp.sum(
        jnp.where(lanes == si, jnp.float32(1.0), jnp.float32(0.0)),
        axis=0, keepdims=True)
    oi = obj_idx_ref[...]
    cnt_o_ref[...] += jnp.sum(
        jnp.where(lanes == oi, jnp.float32(1.0), jnp.float32(0.0)),
        axis=0, keepdims=True)


def _scan_excl(counts):
    lane = jax.lax.broadcasted_iota(jnp.int32, (1, _N), 1)
    incl = counts
    sh = 1
    while sh < _N:
        rolled = pltpu.roll(incl, sh, 1)
        incl = incl + jnp.where(lane >= sh, rolled, jnp.float32(0.0))
        sh *= 2
    return incl - counts


def _split_k(counts):
    offs = _scan_excl(counts)
    k = jnp.full((1, _N), _NO_SPLIT, jnp.float32)
    for b in _BOUNDS:
        fb = jnp.float32(b)
        inwin = (offs < fb) & (fb < offs + counts)
        k = jnp.where(inwin, fb - offs, k)
    return k.astype(jnp.int32)


def _plan_body(cnt_s_ref, cnt_o_ref, k_s_ref, k_o_ref, cnt_ref):
    cs = cnt_s_ref[...]
    co = cnt_o_ref[...]
    k_s_ref[...] = _split_k(cs)
    k_o_ref[...] = _split_k(co)
    cnt_ref[...] = (cs + co).reshape(_N, 1)


def _scatter_body(sub_idx_ref, obj_idx_ref, k_s_ref, k_o_ref,
                  x_s_ref, x_o_ref, sums_ref, acc_ref, run_s_ref, run_o_ref):
    p = pl.program_id(0)

    @pl.when(p == 0)
    def _():
        acc_ref[...] = jnp.zeros_like(acc_ref)

        def zero(i, carry):
            run_s_ref[i] = 0
            run_o_ref[i] = 0
            return carry
        jax.lax.fori_loop(0, _N, zero, 0)

    base = p * _SCAT_CHUNK

    def body(r, carry):
        s = sub_idx_ref[base + r]
        c = run_s_ref[s]
        t = s + jnp.where(c >= k_s_ref[s], _N, 0)
        acc_ref[pl.ds(t, 1), :] += x_s_ref[pl.ds(r, 1), :]
        run_s_ref[s] = c + 1
        o = obj_idx_ref[base + r]
        c2 = run_o_ref[o]
        t2 = o + jnp.where(c2 >= k_o_ref[o], _N, 0) + 2 * _N
        acc_ref[pl.ds(t2, 1), :] += x_o_ref[pl.ds(r, 1), :]
        run_o_ref[o] = c2 + 1
        return carry
    jax.lax.fori_loop(0, _SCAT_CHUNK, body, 0)

    @pl.when(p == pl.num_programs(0) - 1)
    def _():
        sum_s = acc_ref[0:_N, :] + acc_ref[_N:2 * _N, :]
        sum_o = acc_ref[2 * _N:3 * _N, :] + acc_ref[3 * _N:4 * _N, :]
        sums_ref[...] = sum_s + sum_o


def _obj_head_body(sums_ref, cnt_ref, scores_ref, pred_ref):
    refine = sums_ref[...] / jnp.maximum(cnt_ref[...], jnp.float32(1.0))
    m = jnp.max(refine, axis=-1, keepdims=True)
    e = jnp.exp(refine - m)
    e256 = jnp.concatenate(
        [e, jnp.zeros((_N, 256 - _C_OBJ), jnp.float32)], axis=1)
    v01 = e256[:, :128] + e256[:, 128:]
    acc = v01[:, 0:8]
    for k in range(1, 16):
        acc = acc + v01[:, 8 * k:8 * k + 8]
    a4 = acc[:, 0:4] + acc[:, 4:8]
    a2 = a4[:, 0:2] + a4[:, 2:4]
    s = a2[:, 0:1] + a2[:, 1:2]
    prob = e / s
    p1 = prob[:, 1:]
    scores_ref[...] = jnp.max(p1, axis=1).reshape(_N, 1)
    pred_ref[...] = (jnp.argmax(p1, axis=1).astype(jnp.int32) + 1).reshape(_N, 1)


def _rel_head_body(x_ref, sub_idx_ref, obj_idx_ref, t_ref,
                   prob_ref, cls_ref, triple_ref):
    x = x_ref[...]
    m = jnp.max(x, axis=-1, keepdims=True)
    e = jnp.exp(x - m)
    e56 = jnp.concatenate(
        [e, jnp.zeros((_CNT_BLK, 56 - _C_REL), jnp.float32)], axis=1)
    acc = e56[:, 0:8]
    for k in range(1, 7):
        acc = acc + e56[:, 8 * k:8 * k + 8]
    a4 = acc[:, 0:4] + acc[:, 4:8]
    a2 = a4[:, 0:2] + a4[:, 2:4]
    s = a2[:, 0:1] + a2[:, 1:2]
    prob = e / s
    prob_ref[...] = prob
    p1 = prob[:, 1:]
    rs = jnp.max(p1, axis=1)
    cls_ref[...] = (jnp.argmax(p1, axis=1).astype(jnp.int32) + 1).reshape(
        _CNT_BLK, 1)
    lanes = jax.lax.broadcasted_iota(jnp.int32, (_CNT_BLK, _N), 1)
    table = t_ref[...]
    s0 = jnp.sum(jnp.where(lanes == sub_idx_ref[...], table,
                           jnp.float32(0.0)), axis=1)
    s1 = jnp.sum(jnp.where(lanes == obj_idx_ref[...], table,
                           jnp.float32(0.0)), axis=1)
    triple_ref[...] = ((rs * s0) * s1).reshape(_CNT_BLK, 1)


def kernel(rel_logits, sub_logits, obj_logits, rel_pair_idx, bbox):
    del bbox
    sub_ind = rel_pair_idx[:, 0:1]
    obj_ind = rel_pair_idx[:, 1:2]

    cnt_s, cnt_o = pl.pallas_call(
        _count_body,
        grid=(_R // _CNT_BLK,),
        in_specs=[pl.BlockSpec((_CNT_BLK, 1), lambda i: (i, 0)),
                  pl.BlockSpec((_CNT_BLK, 1), lambda i: (i, 0))],
        out_specs=[pl.BlockSpec((1, _N), lambda i: (0, 0)),
                   pl.BlockSpec((1, _N), lambda i: (0, 0))],
        out_shape=[jax.ShapeDtypeStruct((1, _N), jnp.float32)] * 2,
    )(sub_ind, obj_ind)

    k_s, k_o, cnt = pl.pallas_call(
        _plan_body,
        out_shape=[jax.ShapeDtypeStruct((1, _N), jnp.int32),
                   jax.ShapeDtypeStruct((1, _N), jnp.int32),
                   jax.ShapeDtypeStruct((_N, 1), jnp.float32)],
    )(cnt_s, cnt_o)

    sums = pl.pallas_call(
        _scatter_body,
        grid=(_R // _SCAT_CHUNK,),
        in_specs=[
            pl.BlockSpec(memory_space=pltpu.SMEM),
            pl.BlockSpec(memory_space=pltpu.SMEM),
            pl.BlockSpec(memory_space=pltpu.SMEM),
            pl.BlockSpec(memory_space=pltpu.SMEM),
            pl.BlockSpec((_SCAT_CHUNK, _C_OBJ), lambda i: (i, 0)),
            pl.BlockSpec((_SCAT_CHUNK, _C_OBJ), lambda i: (i, 0)),
        ],
        out_specs=pl.BlockSpec((_N, _C_OBJ), lambda i: (0, 0)),
        out_shape=jax.ShapeDtypeStruct((_N, _C_OBJ), jnp.float32),
        scratch_shapes=[pltpu.VMEM((4 * _N, _C_OBJ), jnp.float32),
                        pltpu.SMEM((_N,), jnp.int32),
                        pltpu.SMEM((_N,), jnp.int32)],
    )(sub_ind.reshape(_R), obj_ind.reshape(_R),
      k_s.reshape(_N), k_o.reshape(_N), sub_logits, obj_logits)

    obj_scores, obj_pred = pl.pallas_call(
        _obj_head_body,
        out_shape=[jax.ShapeDtypeStruct((_N, 1), jnp.float32),
                   jax.ShapeDtypeStruct((_N, 1), jnp.int32)],
    )(sums, cnt)
    obj_scores = obj_scores.reshape(_N)
    obj_pred = obj_pred.reshape(_N)

    rel_class_prob, rel_class, triple = pl.pallas_call(
        _rel_head_body,
        grid=(_R // _CNT_BLK,),
        in_specs=[pl.BlockSpec((_CNT_BLK, _C_REL), lambda i: (i, 0)),
                  pl.BlockSpec((_CNT_BLK, 1), lambda i: (i, 0)),
                  pl.BlockSpec((_CNT_BLK, 1), lambda i: (i, 0)),
                  pl.BlockSpec((1, _N), lambda i: (0, 0))],
        out_specs=[pl.BlockSpec((_CNT_BLK, _C_REL), lambda i: (i, 0)),
                   pl.BlockSpec((_CNT_BLK, 1), lambda i: (i, 0)),
                   pl.BlockSpec((_CNT_BLK, 1), lambda i: (i, 0))],
        out_shape=[jax.ShapeDtypeStruct((_R, _C_REL), jnp.float32),
                   jax.ShapeDtypeStruct((_R, 1), jnp.int32),
                   jax.ShapeDtypeStruct((_R, 1), jnp.float32)],
    )(rel_logits, sub_ind, obj_ind, obj_scores.reshape(1, _N))

    triple = triple.reshape(_R)
    sorting_idx = jnp.argsort(-triple)
    rel_pair_sorted = rel_pair_idx[sorting_idx]
    rel_class_prob_sorted = rel_class_prob[sorting_idx]
    rel_labels = rel_class.reshape(_R)[sorting_idx]
    return (obj_pred, obj_scores, rel_pair_sorted,
            rel_class_prob_sorted, rel_labels)
